# baseline (speedup 1.0000x reference)
import functools

import jax
import jax.numpy as jnp
from jax import lax
from jax.experimental import pallas as pl
from jax.experimental.pallas import tpu as pltpu
from jax.experimental.pallas import tpu_sc as plsc

N = 10000
E = 320000
D = 128

NC = 2
NS = 16
NW = NC * NS

NPAD = 10112
STRIPE = NPAD // NS

EPT = E // NW
KA = 80
NKA = EPT // KA

KS = 8
NKS = EPT // KS
NBS = 5

_mesh = plsc.VectorSubcoreMesh(
    core_axis_name="c", subcore_axis_name="s", num_cores=NC, num_subcores=NS
)


@functools.partial(
    pl.kernel,
    out_type=(
        jax.ShapeDtypeStruct((NC, NPAD, D), jnp.float32),
        jax.ShapeDtypeStruct((NC, NPAD), jnp.float32),
    ),
    mesh=_mesh,
    scratch_types=[
        pltpu.VMEM_SHARED((NPAD, D), jnp.float32),
        pltpu.VMEM_SHARED((NPAD,), jnp.float32),
        pltpu.VMEM((KA, D), jnp.float32),
        pltpu.VMEM((KA, D), jnp.float32),
        pltpu.VMEM((KA, D), jnp.float32),
        pltpu.VMEM((KA, D), jnp.float32),
        pltpu.VMEM((KA,), jnp.int32),
        pltpu.VMEM((KA,), jnp.int32),
        pltpu.VMEM((KA,), jnp.int32),
        pltpu.VMEM((KA,), jnp.int32),
        pltpu.VMEM((KA,), jnp.int32),
        pltpu.VMEM((KA,), jnp.int32),
        pltpu.VMEM((KA,), jnp.int32),
        pltpu.VMEM((KA,), jnp.int32),
        pltpu.VMEM((KA,), jnp.float32),
        pltpu.SemaphoreType.DMA,
        pltpu.SemaphoreType.DMA,
        pltpu.SemaphoreType.DMA,
        pltpu.SemaphoreType.DMA,
        pltpu.SemaphoreType.DMA,
        pltpu.SemaphoreType.DMA,
        pltpu.SemaphoreType.DMA,
        pltpu.SemaphoreType.DMA,
        pltpu.SemaphoreType.DMA,
        pltpu.SemaphoreType.DMA,
        pltpu.SemaphoreType.DMA,
        pltpu.SemaphoreType.DMA,
    ],
)
def _sc_aggregate(feat, srcl, dstl, z2d, z1d,
                  out_agg, out_deg,
                  agg_sh, deg_sh, rows0, rows1, rows2, rows3,
                  is0, is1, is2, is3, id0, id1, id2, id3, ones_v,
                  si0, si1, si2, si3, sg0, sg1, sg2, sg3,
                  ss0, ss1, ss2, ss3):
    c = lax.axis_index("c")
    s = lax.axis_index("s")
    wid = c * NS + s
    rows = [rows0, rows1, rows2, rows3]
    isv = [is0, is1, is2, is3]
    idv = [id0, id1, id2, id3]
    sem_i = [si0, si1, si2, si3]
    sem_g = [sg0, sg1, sg2, sg3]
    sem_s = [ss0, ss1, ss2, ss3]

    pltpu.sync_copy(z2d.at[pl.ds(s * STRIPE, STRIPE)],
                    agg_sh.at[pl.ds(s * STRIPE, STRIPE)])

    @pl.when(s == 0)
    def _():
        pltpu.sync_copy(z1d, deg_sh)

    for i in range(KA // 16):
        ones_v[pl.ds(i * 16, 16)] = jnp.full((16,), 1.0, jnp.float32)

    plsc.subcore_barrier()

    base = wid * EPT

    desc_i = [None] * NKA
    desc_g = [None] * NKA
    desc_s = [None] * NKA

    for t in range(NKA + 3):
        b = t % 4
        if t >= 4:
            desc_s[t - 4][0].wait()
            desc_s[t - 4][1].wait()
        if t < NKA:
            off = pl.multiple_of(base + t * KA, 8)
            d1 = pltpu.async_copy(srcl.at[pl.ds(off, KA)], isv[b], sem_i[b])
            d2 = pltpu.async_copy(dstl.at[pl.ds(off, KA)], idv[b], sem_i[b])
            desc_i[t] = (d1, d2)
        if 0 <= t - 1 < NKA:
            g = t - 1
            bg = g % 4
            desc_i[g][0].wait()
            desc_i[g][1].wait()
            desc_g[g] = pltpu.async_copy(feat.at[isv[bg]], rows[bg], sem_g[bg])
        if 0 <= t - 3 < NKA:
            sc = t - 3
            bs = sc % 4
            desc_g[sc].wait()
            d1 = pltpu.async_copy(rows[bs], agg_sh.at[idv[bs]], sem_s[bs],
                                  add=True)
            d2 = pltpu.async_copy(ones_v, deg_sh.at[idv[bs]], sem_s[bs],
                                  add=True)
            desc_s[sc] = (d1, d2)
    desc_s[NKA - 1][0].wait()
    desc_s[NKA - 1][1].wait()

    plsc.subcore_barrier()

    pltpu.sync_copy(agg_sh.at[pl.ds(s * STRIPE, STRIPE)],
                    out_agg.at[c, pl.ds(s * STRIPE, STRIPE)])

    @pl.when(s == 0)
    def _():
        pltpu.sync_copy(deg_sh, out_deg.at[c])


@functools.partial(
    pl.kernel,
    out_type=jax.ShapeDtypeStruct((E, 16), jnp.float32),
    mesh=_mesh,
    scratch_types=[
        pltpu.VMEM((NKS // NBS, NBS * KS), jnp.int32),
        pltpu.VMEM((NKS // NBS, NBS * KS), jnp.int32),
        *([pltpu.VMEM((KS, D), jnp.float32)] * NBS),
        *([pltpu.VMEM((KS, D), jnp.float32)] * NBS),
        *([pltpu.VMEM((NBS * KS, 16), jnp.float32)] * 2),
        *([pltpu.SemaphoreType.DMA] * (2 * NBS + 2)),
    ],
)
def _sc_score(h, src_s, dst_s, out_p, isv_all, idv_all, *bufs):
    c = lax.axis_index("c")
    s = lax.axis_index("s")
    wid = c * NS + s
    base = wid * EPT
    hs = list(bufs[0:NBS])
    hd = list(bufs[NBS:2 * NBS])
    pg = list(bufs[2 * NBS:2 * NBS + 2])
    sem_a = list(bufs[2 * NBS + 2:3 * NBS + 2])
    sem_b = list(bufs[3 * NBS + 2:4 * NBS + 2])
    sem_o = list(bufs[4 * NBS + 2:4 * NBS + 4])
    NB = NBS

    pltpu.sync_copy(src_s.at[wid], isv_all)
    pltpu.sync_copy(dst_s.at[wid], idv_all)

    def compute(hsr, hdr, pr, row0):
        for e in range(KS):
            m = [hsr[e, pl.ds(j * 16, 16)] * hdr[e, pl.ds(j * 16, 16)]
                 for j in range(D // 16)]
            while len(m) > 1:
                m = [m[i] + m[i + 1] for i in range(0, len(m), 2)]
            pr[row0 + e, :] = m[0]

    NGRP = NKS // NB
    GE = NB * KS

    for b in range(NB):
        pltpu.async_copy(h.at[isv_all.at[0, pl.ds(b * KS, KS)]],
                         hs[b], sem_a[b])
        pltpu.async_copy(h.at[idv_all.at[0, pl.ds(b * KS, KS)]],
                         hd[b], sem_b[b])

    def step(j, b, pgr):
        pltpu.make_async_copy(h.at[pl.ds(0, KS)], hs[b], sem_a[b]).wait()
        pltpu.make_async_copy(h.at[pl.ds(0, KS)], hd[b], sem_b[b]).wait()

        compute(hs[b], hd[b], pgr, b * KS)

        @pl.when(j + 1 < NGRP)
        def _():
            jp1 = j + 1
            pltpu.async_copy(h.at[isv_all.at[jp1, pl.ds(b * KS, KS)]],
                             hs[b], sem_a[b])
            pltpu.async_copy(h.at[idv_all.at[jp1, pl.ds(b * KS, KS)]],
                             hd[b], sem_b[b])

    def do_group(j, gb, gp):
        @pl.when(gp >= 1)
        def _():
            pltpu.make_async_copy(
                pg[gb], out_p.at[pl.ds(0, GE)], sem_o[gb]).wait()

        for b in range(NB):
            step(j, b, pg[gb])

        off = pl.multiple_of(base + j * GE, 8)
        pltpu.async_copy(pg[gb], out_p.at[pl.ds(off, GE)], sem_o[gb])

    def pair_body(gp, carry):
        do_group(2 * gp, 0, gp)
        do_group(2 * gp + 1, 1, gp)
        return carry

    lax.fori_loop(0, NGRP // 2, pair_body, 0)

    for gb in range(2):
        pltpu.make_async_copy(pg[gb], out_p.at[pl.ds(0, GE)], sem_o[gb]).wait()


BN = 1000


def _dense_body(x_ref, a0_ref, a1_ref, dg_ref, ws_ref, wn_ref, b_ref, o_ref):
    a = a0_ref[0] + a1_ref[0]
    hn = a * (1.0 / dg_ref[...])
    h = jnp.dot(x_ref[...], ws_ref[...], preferred_element_type=jnp.float32)
    h = h + jnp.dot(hn, wn_ref[...], preferred_element_type=jnp.float32)
    o_ref[...] = h + b_ref[...]


def _dense(x, agg_p, degsum2, ws_t, wn_t, b):
    return pl.pallas_call(
        _dense_body,
        grid=(N // BN,),
        in_specs=[
            pl.BlockSpec((BN, D), lambda i: (i, 0)),
            pl.BlockSpec((1, BN, D), lambda i: (0, i, 0)),
            pl.BlockSpec((1, BN, D), lambda i: (1, i, 0)),
            pl.BlockSpec((BN, 1), lambda i: (i, 0)),
            pl.BlockSpec((D, D), lambda i: (0, 0)),
            pl.BlockSpec((D, D), lambda i: (0, 0)),
            pl.BlockSpec((1, D), lambda i: (0, 0)),
        ],
        out_specs=pl.BlockSpec((BN, D), lambda i: (i, 0)),
        out_shape=jax.ShapeDtypeStruct((N, D), jnp.float32),
    )(x, agg_p, agg_p, degsum2, ws_t, wn_t, b)


PROWS = E * 16 // D
BR = 4000


def _finish_body(p_ref, g_ref, o_ref, mn_ref, mx_ref):
    ph = pl.program_id(0)
    i = pl.program_id(1)
    s = jnp.dot(p_ref[...], g_ref[...], preferred_element_type=jnp.float32)

    @pl.when(ph == 0)
    def _():
        m = jnp.min(s)
        mm = jnp.max(s)

        @pl.when(i == 0)
        def _():
            mn_ref[0, 0] = m
            mx_ref[0, 0] = mm

        @pl.when(i > 0)
        def _():
            mn_ref[0, 0] = jnp.minimum(mn_ref[0, 0], m)
            mx_ref[0, 0] = jnp.maximum(mx_ref[0, 0], mm)

    @pl.when(ph == 1)
    def _():
        scale = 1.0 / (mx_ref[0, 0] - mn_ref[0, 0])
        o_ref[...] = lax.slice((s - mn_ref[0, 0]) * scale, (0, 0), (BR, 8))


def _finish(p2, g):
    return pl.pallas_call(
        _finish_body,
        grid=(2, PROWS // BR),
        in_specs=[
            pl.BlockSpec((BR, D), lambda p, i: (i, 0)),
            pl.BlockSpec((D, D), lambda p, i: (0, 0)),
        ],
        out_specs=pl.BlockSpec((BR, 8), lambda p, i: (i, 0)),
        out_shape=jax.ShapeDtypeStruct((PROWS, 8), jnp.float32),
        scratch_shapes=[
            pltpu.SMEM((1, 1), jnp.float32),
            pltpu.SMEM((1, 1), jnp.float32),
        ],
    )(p2, g)


@jax.jit
def kernel(x, edge_index, W1_self, W1_neigh, b1, W2_self, W2_neigh, b2):
    src = edge_index[0]
    dst = edge_index[1]
    e3 = edge_index.reshape(2, NW, EPT)
    main_s = e3.reshape(2, NW, NKS // NBS, NBS * KS)

    z2d = jnp.zeros((NPAD, D), jnp.float32)
    z1d = jnp.zeros((NPAD,), jnp.float32)

    agg1, deg = _sc_aggregate(x, src, dst, z2d, z1d)
    degsum2 = jnp.maximum(deg[0] + deg[1], 1.0)[:N, None]

    h1 = _dense(x, agg1, degsum2, W1_self.T, W1_neigh.T, b1[None, :])
    agg2, _ = _sc_aggregate(h1, src, dst, z2d, z1d)
    h2 = _dense(h1, agg2, degsum2, W2_self.T, W2_neigh.T, b2[None, :])

    p = _sc_score(h2, main_s[0], main_s[1])
    p2 = p.reshape(PROWS, D)
    col = jnp.arange(D, dtype=jnp.int32)
    g = (col[:, None] // 16 == col[None, :] % 8).astype(jnp.float32)
    label = _finish(p2, g).reshape(E)
    return label


# speedup vs baseline: 1.1249x; 1.1249x over previous
import functools

import jax
import jax.numpy as jnp
from jax import lax
from jax.experimental import pallas as pl
from jax.experimental.pallas import tpu as pltpu
from jax.experimental.pallas import tpu_sc as plsc

N = 10000
E = 320000
D = 128

NC = 2
NS = 16
NW = NC * NS

NPAD = 10112
STRIPE = NPAD // NS

EPT = E // NW
KA = 80
NKA = EPT // KA

KS = 8
NKS = EPT // KS
NBS = 5

_mesh = plsc.VectorSubcoreMesh(
    core_axis_name="c", subcore_axis_name="s", num_cores=NC, num_subcores=NS
)


@functools.partial(
    pl.kernel,
    out_type=(
        jax.ShapeDtypeStruct((NC, NPAD, D), jnp.float32),
        jax.ShapeDtypeStruct((NC, NPAD), jnp.float32),
    ),
    mesh=_mesh,
    scratch_types=[
        pltpu.VMEM_SHARED((NPAD, D), jnp.float32),
        pltpu.VMEM_SHARED((NPAD,), jnp.float32),
        pltpu.VMEM((KA, D), jnp.float32),
        pltpu.VMEM((KA, D), jnp.float32),
        pltpu.VMEM((KA, D), jnp.float32),
        pltpu.VMEM((KA, D), jnp.float32),
        pltpu.VMEM((KA,), jnp.int32),
        pltpu.VMEM((KA,), jnp.int32),
        pltpu.VMEM((KA,), jnp.int32),
        pltpu.VMEM((KA,), jnp.int32),
        pltpu.VMEM((KA,), jnp.int32),
        pltpu.VMEM((KA,), jnp.int32),
        pltpu.VMEM((KA,), jnp.int32),
        pltpu.VMEM((KA,), jnp.int32),
        pltpu.VMEM((KA,), jnp.float32),
        pltpu.SemaphoreType.DMA,
        pltpu.SemaphoreType.DMA,
        pltpu.SemaphoreType.DMA,
        pltpu.SemaphoreType.DMA,
        pltpu.SemaphoreType.DMA,
        pltpu.SemaphoreType.DMA,
        pltpu.SemaphoreType.DMA,
        pltpu.SemaphoreType.DMA,
        pltpu.SemaphoreType.DMA,
        pltpu.SemaphoreType.DMA,
        pltpu.SemaphoreType.DMA,
        pltpu.SemaphoreType.DMA,
    ],
)
def _sc_aggregate(feat, srcl, dstl, z2d, z1d,
                  out_agg, out_deg,
                  agg_sh, deg_sh, rows0, rows1, rows2, rows3,
                  is0, is1, is2, is3, id0, id1, id2, id3, ones_v,
                  si0, si1, si2, si3, sg0, sg1, sg2, sg3,
                  ss0, ss1, ss2, ss3):
    c = lax.axis_index("c")
    s = lax.axis_index("s")
    wid = c * NS + s
    rows = [rows0, rows1, rows2, rows3]
    isv = [is0, is1, is2, is3]
    idv = [id0, id1, id2, id3]
    sem_i = [si0, si1, si2, si3]
    sem_g = [sg0, sg1, sg2, sg3]
    sem_s = [ss0, ss1, ss2, ss3]

    pltpu.sync_copy(z2d.at[pl.ds(s * STRIPE, STRIPE)],
                    agg_sh.at[pl.ds(s * STRIPE, STRIPE)])

    @pl.when(s == 0)
    def _():
        pltpu.sync_copy(z1d, deg_sh)

    for i in range(KA // 16):
        ones_v[pl.ds(i * 16, 16)] = jnp.full((16,), 1.0, jnp.float32)

    plsc.subcore_barrier()

    base = wid * EPT

    desc_i = [None] * NKA
    desc_g = [None] * NKA
    desc_s = [None] * NKA

    for t in range(NKA + 3):
        b = t % 4
        if t >= 4:
            desc_s[t - 4][0].wait()
            desc_s[t - 4][1].wait()
        if t < NKA:
            off = pl.multiple_of(base + t * KA, 8)
            d1 = pltpu.async_copy(srcl.at[pl.ds(off, KA)], isv[b], sem_i[b])
            d2 = pltpu.async_copy(dstl.at[pl.ds(off, KA)], idv[b], sem_i[b])
            desc_i[t] = (d1, d2)
        if 0 <= t - 1 < NKA:
            g = t - 1
            bg = g % 4
            desc_i[g][0].wait()
            desc_i[g][1].wait()
            desc_g[g] = pltpu.async_copy(feat.at[isv[bg]], rows[bg], sem_g[bg])
        if 0 <= t - 3 < NKA:
            sc = t - 3
            bs = sc % 4
            desc_g[sc].wait()
            d1 = pltpu.async_copy(rows[bs], agg_sh.at[idv[bs]], sem_s[bs],
                                  add=True)
            d2 = pltpu.async_copy(ones_v, deg_sh.at[idv[bs]], sem_s[bs],
                                  add=True)
            desc_s[sc] = (d1, d2)
    desc_s[NKA - 1][0].wait()
    desc_s[NKA - 1][1].wait()

    plsc.subcore_barrier()

    pltpu.sync_copy(agg_sh.at[pl.ds(s * STRIPE, STRIPE)],
                    out_agg.at[c, pl.ds(s * STRIPE, STRIPE)])

    @pl.when(s == 0)
    def _():
        pltpu.sync_copy(deg_sh, out_deg.at[c])


@functools.partial(
    pl.kernel,
    out_type=jax.ShapeDtypeStruct((E, 16), jnp.float32),
    mesh=_mesh,
    scratch_types=[
        pltpu.VMEM((NKS // NBS, NBS * KS), jnp.int32),
        pltpu.VMEM((NKS // NBS, NBS * KS), jnp.int32),
        *([pltpu.VMEM((KS, D), jnp.float32)] * NBS),
        *([pltpu.VMEM((KS, D), jnp.float32)] * NBS),
        *([pltpu.VMEM((KS, 16), jnp.float32)] * NBS),
        *([pltpu.SemaphoreType.DMA] * (3 * NBS)),
    ],
)
def _sc_score(h, src_s, dst_s, out_p, isv_all, idv_all, *bufs):
    c = lax.axis_index("c")
    s = lax.axis_index("s")
    wid = c * NS + s
    base = wid * EPT
    hs = list(bufs[0:NBS])
    hd = list(bufs[NBS:2 * NBS])
    p = list(bufs[2 * NBS:3 * NBS])
    sem_a = list(bufs[3 * NBS:4 * NBS])
    sem_b = list(bufs[4 * NBS:5 * NBS])
    sem_o = list(bufs[5 * NBS:6 * NBS])
    NB = NBS

    pltpu.sync_copy(src_s.at[wid], isv_all)
    pltpu.sync_copy(dst_s.at[wid], idv_all)

    def compute(hsr, hdr, pr, k):
        for e in range(k):
            m = [hsr[e, pl.ds(j * 16, 16)] * hdr[e, pl.ds(j * 16, 16)]
                 for j in range(D // 16)]
            while len(m) > 1:
                m = [m[i] + m[i + 1] for i in range(0, len(m), 2)]
            pr[e, :] = m[0]

    NGRP = NKS // NB

    for b in range(NB):
        pltpu.async_copy(h.at[isv_all.at[0, pl.ds(b * KS, KS)]],
                         hs[b], sem_a[b])
        pltpu.async_copy(h.at[idv_all.at[0, pl.ds(b * KS, KS)]],
                         hd[b], sem_b[b])

    def step(j, b):
        i = NB * j + b
        @pl.when(j >= 1)
        def _():
            pltpu.make_async_copy(
                p[b], out_p.at[pl.ds(0, KS)], sem_o[b]).wait()

        pltpu.make_async_copy(h.at[pl.ds(0, KS)], hs[b], sem_a[b]).wait()
        pltpu.make_async_copy(h.at[pl.ds(0, KS)], hd[b], sem_b[b]).wait()

        compute(hs[b], hd[b], p[b], KS)

        off = pl.multiple_of(base + i * KS, 8)
        pltpu.async_copy(p[b], out_p.at[pl.ds(off, KS)], sem_o[b])

        @pl.when(j + 1 < NGRP)
        def _():
            jp1 = j + 1
            pltpu.async_copy(h.at[isv_all.at[jp1, pl.ds(b * KS, KS)]],
                             hs[b], sem_a[b])
            pltpu.async_copy(h.at[idv_all.at[jp1, pl.ds(b * KS, KS)]],
                             hd[b], sem_b[b])

    def group_body(j, carry):
        for b in range(NB):
            step(j, b)
        return carry

    lax.fori_loop(0, NGRP, group_body, 0)

    for b in range(NB):
        pltpu.make_async_copy(p[b], out_p.at[pl.ds(0, KS)], sem_o[b]).wait()


BN = 1000


def _dense_body(x_ref, a0_ref, a1_ref, dg_ref, ws_ref, wn_ref, b_ref, o_ref):
    a = a0_ref[0] + a1_ref[0]
    hn = a * (1.0 / dg_ref[...])
    h = jnp.dot(x_ref[...], ws_ref[...], preferred_element_type=jnp.float32)
    h = h + jnp.dot(hn, wn_ref[...], preferred_element_type=jnp.float32)
    o_ref[...] = h + b_ref[...]


def _dense(x, agg_p, degsum2, ws_t, wn_t, b):
    return pl.pallas_call(
        _dense_body,
        grid=(N // BN,),
        in_specs=[
            pl.BlockSpec((BN, D), lambda i: (i, 0)),
            pl.BlockSpec((1, BN, D), lambda i: (0, i, 0)),
            pl.BlockSpec((1, BN, D), lambda i: (1, i, 0)),
            pl.BlockSpec((BN, 1), lambda i: (i, 0)),
            pl.BlockSpec((D, D), lambda i: (0, 0)),
            pl.BlockSpec((D, D), lambda i: (0, 0)),
            pl.BlockSpec((1, D), lambda i: (0, 0)),
        ],
        out_specs=pl.BlockSpec((BN, D), lambda i: (i, 0)),
        out_shape=jax.ShapeDtypeStruct((N, D), jnp.float32),
    )(x, agg_p, agg_p, degsum2, ws_t, wn_t, b)


PROWS = E * 16 // D
BR = 4000


def _finish_body(p_ref, g_ref, o_ref, mn_ref, mx_ref):
    ph = pl.program_id(0)
    i = pl.program_id(1)
    s = jnp.dot(p_ref[...], g_ref[...], preferred_element_type=jnp.float32)

    @pl.when(ph == 0)
    def _():
        m = jnp.min(s)
        mm = jnp.max(s)

        @pl.when(i == 0)
        def _():
            mn_ref[0, 0] = m
            mx_ref[0, 0] = mm

        @pl.when(i > 0)
        def _():
            mn_ref[0, 0] = jnp.minimum(mn_ref[0, 0], m)
            mx_ref[0, 0] = jnp.maximum(mx_ref[0, 0], mm)

    @pl.when(ph == 1)
    def _():
        scale = 1.0 / (mx_ref[0, 0] - mn_ref[0, 0])
        o_ref[...] = lax.slice((s - mn_ref[0, 0]) * scale, (0, 0), (BR, 8))


def _finish(p2, g):
    return pl.pallas_call(
        _finish_body,
        grid=(2, PROWS // BR),
        in_specs=[
            pl.BlockSpec((BR, D), lambda p, i: (i, 0)),
            pl.BlockSpec((D, D), lambda p, i: (0, 0)),
        ],
        out_specs=pl.BlockSpec((BR, 8), lambda p, i: (i, 0)),
        out_shape=jax.ShapeDtypeStruct((PROWS, 8), jnp.float32),
        scratch_shapes=[
            pltpu.SMEM((1, 1), jnp.float32),
            pltpu.SMEM((1, 1), jnp.float32),
        ],
    )(p2, g)


@jax.jit
def kernel(x, edge_index, W1_self, W1_neigh, b1, W2_self, W2_neigh, b2):
    src = edge_index[0]
    dst = edge_index[1]
    e3 = edge_index.reshape(2, NW, EPT)
    main_s = e3.reshape(2, NW, NKS // NBS, NBS * KS)

    z2d = jnp.zeros((NPAD, D), jnp.float32)
    z1d = jnp.zeros((NPAD,), jnp.float32)

    agg1, deg = _sc_aggregate(x, src, dst, z2d, z1d)
    degsum2 = jnp.maximum(deg[0] + deg[1], 1.0)[:N, None]

    h1 = _dense(x, agg1, degsum2, W1_self.T, W1_neigh.T, b1[None, :])
    agg2, _ = _sc_aggregate(h1, src, dst, z2d, z1d)
    h2 = _dense(h1, agg2, degsum2, W2_self.T, W2_neigh.T, b2[None, :])

    p = _sc_score(h2, main_s[0], main_s[1])
    p2 = p.reshape(PROWS, D)
    col = jnp.arange(D, dtype=jnp.int32)
    g = (col[:, None] // 16 == col[None, :] % 8).astype(jnp.float32)
    label = _finish(p2, g).reshape(E)
    return label
